# baseline (speedup 1.0000x reference)
import functools

import jax
import jax.numpy as jnp
from jax.experimental import pallas as pl
from jax.experimental.pallas import tpu as pltpu

E = 8


def _body(idx_ref, zs_ref, na_ref, w_hbm, hc_ref, x_ref, win_ref, wout_ref,
          o_ref, wbuf, acc_ref, sems, *, bond, n_embd):
    na = na_ref[0]
    nsteps = (na + E - 1) // E

    def issue(step, slot):
        for e in range(E):
            j = idx_ref[step * E + e]
            pltpu.make_async_copy(
                w_hbm.at[pl.ds(j, 1)], wbuf.at[slot, pl.ds(e, 1)],
                sems.at[slot]).start()

    def wait(step, slot):
        for e in range(E):
            j = idx_ref[step * E + e]
            pltpu.make_async_copy(
                w_hbm.at[pl.ds(j, 1)], wbuf.at[slot, pl.ds(e, 1)],
                sems.at[slot]).wait()

    acc_ref[...] = jnp.zeros_like(acc_ref)

    @pl.when(nsteps > 0)
    def _prologue():
        issue(0, 0)

    def loop(step, carry):
        slot = jax.lax.rem(step, 2)

        @pl.when(step + 1 < nsteps)
        def _next():
            issue(step + 1, 1 - slot)

        wait(step, slot)
        partial = jnp.zeros((1, bond), jnp.float32)
        for e in range(E):
            k = step * E + e
            j = idx_ref[k]
            zv = zs_ref[k]
            y = hc_ref[pl.ds(j, 1), :] * zv
            partial += jax.lax.dot_general(
                y, wbuf[slot, e], (((1,), (1,)), ((), ())),
                preferred_element_type=jnp.float32)
        acc_ref[...] += partial
        return carry

    jax.lax.fori_loop(0, nsteps, loop, 0, unroll=False)

    h = jax.lax.dot_general(
        x_ref[...], win_ref[...], (((1,), (1,)), ((), ())),
        preferred_element_type=jnp.float32)
    outv = h + acc_ref[...]
    o_ref[...] = jax.lax.dot_general(
        outv, wout_ref[...], (((1,), (1,)), ((), ())),
        preferred_element_type=jnp.float32)


def kernel(x, pos, hidden_cache, W_in, W_out, W_edges, gate_logits):
    del pos
    POS, BOND = hidden_cache.shape
    N = x.shape[0]

    z = jnp.clip(gate_logits, 0.0, 1.0)
    active = z > 0.0
    slot = jnp.arange(POS, dtype=jnp.int32)
    p = jnp.cumsum(active.astype(jnp.int32)) - 1
    na = p[-1] + 1
    tgt = jnp.where(active, p, POS)
    order = jnp.zeros(POS, jnp.int32).at[tgt].set(slot, mode="drop")
    zvals = jnp.zeros(POS, jnp.float32).at[tgt].set(z, mode="drop")
    last = order[jnp.maximum(na - 1, 0)]
    idx = jnp.where(slot < na, order, last)
    zs = jnp.where(slot < na, zvals, 0.0)
    na_arr = jnp.full((1,), na, jnp.int32)

    out = pl.pallas_call(
        functools.partial(_body, bond=BOND, n_embd=N),
        in_specs=[
            pl.BlockSpec(memory_space=pltpu.SMEM),
            pl.BlockSpec(memory_space=pltpu.SMEM),
            pl.BlockSpec(memory_space=pltpu.SMEM),
            pl.BlockSpec(memory_space=pl.ANY),
            pl.BlockSpec(memory_space=pltpu.VMEM),
            pl.BlockSpec(memory_space=pltpu.VMEM),
            pl.BlockSpec(memory_space=pltpu.VMEM),
            pl.BlockSpec(memory_space=pltpu.VMEM),
        ],
        out_specs=pl.BlockSpec(memory_space=pltpu.VMEM),
        out_shape=jax.ShapeDtypeStruct((1, N), jnp.float32),
        scratch_shapes=[
            pltpu.VMEM((2, E, BOND, BOND), jnp.float32),
            pltpu.VMEM((1, BOND), jnp.float32),
            pltpu.SemaphoreType.DMA((2,)),
        ],
    )(idx, zs, na_arr, W_edges, hidden_cache, x.reshape(1, N), W_in, W_out)
    return out.reshape(N)


# speedup vs baseline: 1.4556x; 1.4556x over previous
import functools

import jax
import jax.numpy as jnp
from jax.experimental import pallas as pl
from jax.experimental.pallas import tpu as pltpu

E = 8
NBUF = 4


def _body(idx_ref, zs_ref, na_ref, w_hbm, hc_ref, x_ref, win_ref, wout_ref,
          o_ref, wbuf, acc_ref, sems, *, bond, n_embd):
    na = na_ref[0]
    nsteps = (na + E - 1) // E

    def issue(step, slot):
        for e in range(E):
            j = idx_ref[step * E + e]
            pltpu.make_async_copy(
                w_hbm.at[pl.ds(j, 1)], wbuf.at[slot, pl.ds(e, 1)],
                sems.at[slot]).start()

    def wait(step, slot):
        for e in range(E):
            j = idx_ref[step * E + e]
            pltpu.make_async_copy(
                w_hbm.at[pl.ds(j, 1)], wbuf.at[slot, pl.ds(e, 1)],
                sems.at[slot]).wait()

    acc_ref[...] = jnp.zeros_like(acc_ref)

    for k in range(NBUF - 1):
        @pl.when(k < nsteps)
        def _prologue():
            issue(k, k)

    def loop(step, carry):
        slot = jax.lax.rem(step, NBUF)

        @pl.when(step + NBUF - 1 < nsteps)
        def _next():
            issue(step + NBUF - 1, jax.lax.rem(step + NBUF - 1, NBUF))

        wait(step, slot)
        partial = jnp.zeros((1, bond), jnp.float32)
        for e in range(E):
            k = step * E + e
            j = idx_ref[k]
            zv = zs_ref[k]
            y = hc_ref[pl.ds(j, 1), :] * zv
            partial += jax.lax.dot_general(
                y, wbuf[slot, e], (((1,), (1,)), ((), ())),
                preferred_element_type=jnp.float32)
        acc_ref[...] += partial
        return carry

    jax.lax.fori_loop(0, nsteps, loop, 0, unroll=False)

    h = jax.lax.dot_general(
        x_ref[...], win_ref[...], (((1,), (1,)), ((), ())),
        preferred_element_type=jnp.float32)
    outv = h + acc_ref[...]
    o_ref[...] = jax.lax.dot_general(
        outv, wout_ref[...], (((1,), (1,)), ((), ())),
        preferred_element_type=jnp.float32)


def kernel(x, pos, hidden_cache, W_in, W_out, W_edges, gate_logits):
    del pos
    POS, BOND = hidden_cache.shape
    N = x.shape[0]

    z = jnp.clip(gate_logits, 0.0, 1.0)
    active = z > 0.0
    slot = jnp.arange(POS, dtype=jnp.int32)
    p = jnp.cumsum(active.astype(jnp.int32)) - 1
    na = p[-1] + 1
    tgt = jnp.where(active, p, POS)
    order = jnp.zeros(POS, jnp.int32).at[tgt].set(slot, mode="drop")
    zvals = jnp.zeros(POS, jnp.float32).at[tgt].set(z, mode="drop")
    last = order[jnp.maximum(na - 1, 0)]
    idx = jnp.where(slot < na, order, last)
    zs = jnp.where(slot < na, zvals, 0.0)
    na_arr = jnp.full((1,), na, jnp.int32)

    out = pl.pallas_call(
        functools.partial(_body, bond=BOND, n_embd=N),
        in_specs=[
            pl.BlockSpec(memory_space=pltpu.SMEM),
            pl.BlockSpec(memory_space=pltpu.SMEM),
            pl.BlockSpec(memory_space=pltpu.SMEM),
            pl.BlockSpec(memory_space=pl.ANY),
            pl.BlockSpec(memory_space=pltpu.VMEM),
            pl.BlockSpec(memory_space=pltpu.VMEM),
            pl.BlockSpec(memory_space=pltpu.VMEM),
            pl.BlockSpec(memory_space=pltpu.VMEM),
        ],
        out_specs=pl.BlockSpec(memory_space=pltpu.VMEM),
        out_shape=jax.ShapeDtypeStruct((1, N), jnp.float32),
        scratch_shapes=[
            pltpu.VMEM((NBUF, E, BOND, BOND), jnp.float32),
            pltpu.VMEM((1, BOND), jnp.float32),
            pltpu.SemaphoreType.DMA((NBUF,)),
        ],
    )(idx, zs, na_arr, W_edges, hidden_cache, x.reshape(1, N), W_in, W_out)
    return out.reshape(N)


# speedup vs baseline: 1.4583x; 1.0018x over previous
import functools

import jax
import jax.numpy as jnp
from jax.experimental import pallas as pl
from jax.experimental.pallas import tpu as pltpu

E = 8
NBUF = 8


def _body(idx_ref, zs_ref, na_ref, w_hbm, hc_ref, x_ref, win_ref, wout_ref,
          o_ref, wbuf, acc_ref, sems, *, bond, n_embd):
    na = na_ref[0]
    nsteps = (na + E - 1) // E

    def issue(step, slot):
        for e in range(E):
            j = idx_ref[step * E + e]
            pltpu.make_async_copy(
                w_hbm.at[pl.ds(j, 1)], wbuf.at[slot, pl.ds(e, 1)],
                sems.at[slot]).start()

    def wait(step, slot):
        for e in range(E):
            j = idx_ref[step * E + e]
            pltpu.make_async_copy(
                w_hbm.at[pl.ds(j, 1)], wbuf.at[slot, pl.ds(e, 1)],
                sems.at[slot]).wait()

    acc_ref[...] = jnp.zeros_like(acc_ref)

    for k in range(NBUF - 1):
        @pl.when(k < nsteps)
        def _prologue():
            issue(k, k)

    def loop(step, carry):
        slot = jax.lax.rem(step, NBUF)

        @pl.when(step + NBUF - 1 < nsteps)
        def _next():
            issue(step + NBUF - 1, jax.lax.rem(step + NBUF - 1, NBUF))

        wait(step, slot)
        partial = jnp.zeros((1, bond), jnp.float32)
        for e in range(E):
            k = step * E + e
            j = idx_ref[k]
            zv = zs_ref[k]
            y = hc_ref[pl.ds(j, 1), :] * zv
            partial += jax.lax.dot_general(
                y, wbuf[slot, e], (((1,), (1,)), ((), ())),
                preferred_element_type=jnp.float32)
        acc_ref[...] += partial
        return carry

    jax.lax.fori_loop(0, nsteps, loop, 0, unroll=False)

    h = jax.lax.dot_general(
        x_ref[...], win_ref[...], (((1,), (1,)), ((), ())),
        preferred_element_type=jnp.float32)
    outv = h + acc_ref[...]
    o_ref[...] = jax.lax.dot_general(
        outv, wout_ref[...], (((1,), (1,)), ((), ())),
        preferred_element_type=jnp.float32)


def kernel(x, pos, hidden_cache, W_in, W_out, W_edges, gate_logits):
    del pos
    POS, BOND = hidden_cache.shape
    N = x.shape[0]

    z = jnp.clip(gate_logits, 0.0, 1.0)
    active = z > 0.0
    slot = jnp.arange(POS, dtype=jnp.int32)
    p = jnp.cumsum(active.astype(jnp.int32)) - 1
    na = p[-1] + 1
    tgt = jnp.where(active, p, POS)
    order = jnp.zeros(POS, jnp.int32).at[tgt].set(slot, mode="drop")
    zvals = jnp.zeros(POS, jnp.float32).at[tgt].set(z, mode="drop")
    last = order[jnp.maximum(na - 1, 0)]
    idx = jnp.where(slot < na, order, last)
    zs = jnp.where(slot < na, zvals, 0.0)
    na_arr = jnp.full((1,), na, jnp.int32)

    out = pl.pallas_call(
        functools.partial(_body, bond=BOND, n_embd=N),
        in_specs=[
            pl.BlockSpec(memory_space=pltpu.SMEM),
            pl.BlockSpec(memory_space=pltpu.SMEM),
            pl.BlockSpec(memory_space=pltpu.SMEM),
            pl.BlockSpec(memory_space=pl.ANY),
            pl.BlockSpec(memory_space=pltpu.VMEM),
            pl.BlockSpec(memory_space=pltpu.VMEM),
            pl.BlockSpec(memory_space=pltpu.VMEM),
            pl.BlockSpec(memory_space=pltpu.VMEM),
        ],
        out_specs=pl.BlockSpec(memory_space=pltpu.VMEM),
        out_shape=jax.ShapeDtypeStruct((1, N), jnp.float32),
        scratch_shapes=[
            pltpu.VMEM((NBUF, E, BOND, BOND), jnp.float32),
            pltpu.VMEM((1, BOND), jnp.float32),
            pltpu.SemaphoreType.DMA((NBUF,)),
        ],
    )(idx, zs, na_arr, W_edges, hidden_cache, x.reshape(1, N), W_in, W_out)
    return out.reshape(N)


# speedup vs baseline: 1.5186x; 1.0414x over previous
import functools

import jax
import jax.numpy as jnp
from jax.experimental import pallas as pl
from jax.experimental.pallas import tpu as pltpu

E = 8
NBUF = 8


def _body(gl_ref, w_hbm, hc_ref, x_ref, win_ref, wout_ref,
          o_ref, wbuf, acc_ref, idx_ref, zs_ref, sems, *, bond, n_embd, pos):
    def scan(j, p):
        g = gl_ref[j]
        z = jnp.minimum(jnp.maximum(g, 0.0), 1.0)
        idx_ref[p] = j
        zs_ref[p] = z
        return p + (z > 0.0).astype(jnp.int32)

    na = jax.lax.fori_loop(0, pos, scan, 0, unroll=False)

    lastj = jnp.where(na > 0, idx_ref[jnp.maximum(na - 1, 0)], 0)
    for e in range(E):
        idx_ref[na + e] = lastj
        zs_ref[na + e] = 0.0

    nsteps = (na + E - 1) // E

    def issue(step, slot):
        for e in range(E):
            j = idx_ref[step * E + e]
            pltpu.make_async_copy(
                w_hbm.at[pl.ds(j, 1)], wbuf.at[slot, pl.ds(e, 1)],
                sems.at[slot]).start()

    def wait(step, slot):
        for e in range(E):
            j = idx_ref[step * E + e]
            pltpu.make_async_copy(
                w_hbm.at[pl.ds(j, 1)], wbuf.at[slot, pl.ds(e, 1)],
                sems.at[slot]).wait()

    acc_ref[...] = jnp.zeros_like(acc_ref)

    for k in range(NBUF - 1):
        @pl.when(k < nsteps)
        def _prologue():
            issue(k, k)

    def loop(step, carry):
        slot = jax.lax.rem(step, NBUF)

        @pl.when(step + NBUF - 1 < nsteps)
        def _next():
            issue(step + NBUF - 1, jax.lax.rem(step + NBUF - 1, NBUF))

        wait(step, slot)
        partial = jnp.zeros((1, bond), jnp.float32)
        for e in range(E):
            k = step * E + e
            j = idx_ref[k]
            zv = zs_ref[k]
            y = hc_ref[pl.ds(j, 1), :] * zv
            partial += jax.lax.dot_general(
                y, wbuf[slot, e], (((1,), (1,)), ((), ())),
                preferred_element_type=jnp.float32)
        acc_ref[...] += partial
        return carry

    jax.lax.fori_loop(0, nsteps, loop, 0, unroll=False)

    h = jax.lax.dot_general(
        x_ref[...], win_ref[...], (((1,), (1,)), ((), ())),
        preferred_element_type=jnp.float32)
    outv = h + acc_ref[...]
    o_ref[...] = jax.lax.dot_general(
        outv, wout_ref[...], (((1,), (1,)), ((), ())),
        preferred_element_type=jnp.float32)


def kernel(x, pos, hidden_cache, W_in, W_out, W_edges, gate_logits):
    del pos
    POS, BOND = hidden_cache.shape
    N = x.shape[0]

    out = pl.pallas_call(
        functools.partial(_body, bond=BOND, n_embd=N, pos=POS),
        in_specs=[
            pl.BlockSpec(memory_space=pltpu.SMEM),
            pl.BlockSpec(memory_space=pl.ANY),
            pl.BlockSpec(memory_space=pltpu.VMEM),
            pl.BlockSpec(memory_space=pltpu.VMEM),
            pl.BlockSpec(memory_space=pltpu.VMEM),
            pl.BlockSpec(memory_space=pltpu.VMEM),
        ],
        out_specs=pl.BlockSpec(memory_space=pltpu.VMEM),
        out_shape=jax.ShapeDtypeStruct((1, N), jnp.float32),
        scratch_shapes=[
            pltpu.VMEM((NBUF, E, BOND, BOND), jnp.float32),
            pltpu.VMEM((1, BOND), jnp.float32),
            pltpu.SMEM((POS + E,), jnp.int32),
            pltpu.SMEM((POS + E,), jnp.float32),
            pltpu.SemaphoreType.DMA((NBUF,)),
        ],
    )(gate_logits, W_edges, hidden_cache, x.reshape(1, N), W_in, W_out)
    return out.reshape(N)
